# baseline (speedup 1.0000x reference)
import jax
import jax.numpy as jnp
from jax.experimental import pallas as pl
from jax.experimental.pallas import tpu as pltpu

_N = 10000
_BM = 400


def _s1_body(x_ref, w1_ref, out_ref):
    out_ref[...] = jnp.dot(x_ref[...], w1_ref[...])


def _layer1_body(a_ref, s1_ref, b1_ref, w2_ref, out_ref):
    h = jnp.dot(a_ref[...], s1_ref[...]) + b1_ref[...]
    x = jnp.maximum(h, 0.0)
    out_ref[...] = jnp.dot(x, w2_ref[...])


def _layer2_body(a_ref, s2_ref, b2_ref, out_ref):
    o = jnp.dot(a_ref[...], s2_ref[...]) + b2_ref[...]
    m = jnp.max(o, axis=1, keepdims=True)
    e = jnp.exp(o - m)
    out_ref[...] = e / jnp.sum(e, axis=1, keepdims=True)


def kernel(in_feat, adj_mat, W1, b1, W2, b2):
    n, f = in_feat.shape
    h = W1.shape[1]
    c = W2.shape[1]
    b1r = b1.reshape(1, h)
    b2r = b2.reshape(1, c)

    s1 = pl.pallas_call(
        _s1_body,
        out_shape=jax.ShapeDtypeStruct((n, h), jnp.float32),
    )(in_feat, W1)

    grid = (n // _BM,)
    s2 = pl.pallas_call(
        _layer1_body,
        grid=grid,
        in_specs=[
            pl.BlockSpec((_BM, n), lambda i: (i, 0)),
            pl.BlockSpec((n, h), lambda i: (0, 0)),
            pl.BlockSpec((1, h), lambda i: (0, 0)),
            pl.BlockSpec((h, c), lambda i: (0, 0)),
        ],
        out_specs=pl.BlockSpec((_BM, c), lambda i: (i, 0)),
        out_shape=jax.ShapeDtypeStruct((n, c), jnp.float32),
    )(adj_mat, s1, b1r, W2)

    out = pl.pallas_call(
        _layer2_body,
        grid=grid,
        in_specs=[
            pl.BlockSpec((_BM, n), lambda i: (i, 0)),
            pl.BlockSpec((n, c), lambda i: (0, 0)),
            pl.BlockSpec((1, c), lambda i: (0, 0)),
        ],
        out_specs=pl.BlockSpec((_BM, c), lambda i: (i, 0)),
        out_shape=jax.ShapeDtypeStruct((n, c), jnp.float32),
    )(adj_mat, s2, b2r)
    return out


# speedup vs baseline: 1.0561x; 1.0561x over previous
import jax
import jax.numpy as jnp
from jax.experimental import pallas as pl
from jax.experimental.pallas import tpu as pltpu

_BM = 400


def _fused_body(x_ref, a_ref, w1_ref, b1_ref, w2_ref, b2_ref, out_ref,
                s1_ref, s2_ref):
    p = pl.program_id(0)
    i = pl.program_id(1)

    @pl.when(jnp.logical_and(p == 0, i == 0))
    def _():
        s1_ref[...] = jnp.dot(x_ref[...], w1_ref[...])

    @pl.when(p == 0)
    def _():
        h = jnp.dot(a_ref[...], s1_ref[...]) + b1_ref[...]
        x = jnp.maximum(h, 0.0)
        s2_ref[pl.ds(i * _BM, _BM), :] = jnp.dot(x, w2_ref[...])

    @pl.when(p == 1)
    def _():
        o = jnp.dot(a_ref[...], s2_ref[...]) + b2_ref[...]
        m = jnp.max(o, axis=1, keepdims=True)
        e = jnp.exp(o - m)
        out_ref[...] = e / jnp.sum(e, axis=1, keepdims=True)


def kernel(in_feat, adj_mat, W1, b1, W2, b2):
    n, f = in_feat.shape
    h = W1.shape[1]
    c = W2.shape[1]
    b1r = b1.reshape(1, h)
    b2r = b2.reshape(1, c)

    return pl.pallas_call(
        _fused_body,
        grid=(2, n // _BM),
        in_specs=[
            pl.BlockSpec((n, f), lambda p, i: (0, 0)),
            pl.BlockSpec((_BM, n), lambda p, i: (i, 0)),
            pl.BlockSpec((f, h), lambda p, i: (0, 0)),
            pl.BlockSpec((1, h), lambda p, i: (0, 0)),
            pl.BlockSpec((h, c), lambda p, i: (0, 0)),
            pl.BlockSpec((1, c), lambda p, i: (0, 0)),
        ],
        out_specs=pl.BlockSpec((_BM, c), lambda p, i: (i * p, 0)),
        out_shape=jax.ShapeDtypeStruct((n, c), jnp.float32),
        scratch_shapes=[
            pltpu.VMEM((n, h), jnp.float32),
            pltpu.VMEM((n, c), jnp.float32),
        ],
    )(in_feat, adj_mat, W1, b1r, W2, b2r)


# speedup vs baseline: 1.0586x; 1.0024x over previous
import jax
import jax.numpy as jnp
from jax.experimental import pallas as pl
from jax.experimental.pallas import tpu as pltpu

_BM = 400


def _fused_body(x_ref, a_ref, w1_ref, b1_ref, w2_ref, b2_ref, out_ref,
                s1_ref, s2_ref):
    p = pl.program_id(0)
    i = pl.program_id(1)

    @pl.when(jnp.logical_and(p == 0, i == 0))
    def _():
        s1_ref[...] = jnp.dot(x_ref[...], w1_ref[...])

    @pl.when(p == 0)
    def _():
        h = jnp.dot(a_ref[...], s1_ref[...]) + b1_ref[...]
        x = jnp.maximum(h, 0.0)
        s2_ref[pl.ds(i * _BM, _BM), :] = jnp.dot(x, w2_ref[...])

    @pl.when(p == 1)
    def _():
        o = jnp.dot(a_ref[...], s2_ref[...]) + b2_ref[...]
        m = jnp.max(o, axis=1, keepdims=True)
        e = jnp.exp(o - m)
        out_ref[...] = e / jnp.sum(e, axis=1, keepdims=True)


def kernel(in_feat, adj_mat, W1, b1, W2, b2):
    n, f = in_feat.shape
    h = W1.shape[1]
    c = W2.shape[1]
    b1r = b1.reshape(1, h)
    b2r = b2.reshape(1, c)

    nb = n // _BM
    return pl.pallas_call(
        _fused_body,
        grid=(2, nb),
        in_specs=[
            pl.BlockSpec((n, f), lambda p, i: (0, 0)),
            pl.BlockSpec((_BM, n),
                         lambda p, i: (jnp.where(p == 0, i, nb - 1 - i), 0)),
            pl.BlockSpec((f, h), lambda p, i: (0, 0)),
            pl.BlockSpec((1, h), lambda p, i: (0, 0)),
            pl.BlockSpec((h, c), lambda p, i: (0, 0)),
            pl.BlockSpec((1, c), lambda p, i: (0, 0)),
        ],
        out_specs=pl.BlockSpec((_BM, c), lambda p, i: (nb - 1 - i * p, 0)),
        out_shape=jax.ShapeDtypeStruct((n, c), jnp.float32),
        scratch_shapes=[
            pltpu.VMEM((n, h), jnp.float32),
            pltpu.VMEM((n, c), jnp.float32),
        ],
    )(in_feat, adj_mat, W1, b1r, W2, b2r)
